# baseline (speedup 1.0000x reference)
import jax
import jax.numpy as jnp
from jax.experimental import pallas as pl
from jax.experimental.pallas import tpu as pltpu

B, L, D_MODEL = 8, 576, 768
K, D_CODEX, G = 1024, 256, 4
DG = D_CODEX // G
DI = D_MODEL // G
BETA = 0.25


def _body(Z_ref, WpreT_ref, gng_ref, gnb_ref, codex_ref, codexT_ref,
          lng_ref, lnb_ref, WpostT_ref, bpost_ref,
          Zq_ref, loss_ref, probs_ref):
    b = pl.program_id(0)
    Zb = Z_ref[0]

    zq_parts = []
    loss_part = jnp.float32(0.0)
    for g in range(G):
        Zg = Zb[:, g * DI:(g + 1) * DI]
        ze = jnp.dot(Zg, WpreT_ref[g],
                     preferred_element_type=jnp.float32)
        mean = jnp.mean(ze)
        var = jnp.mean((ze - mean) ** 2)
        ze = (ze - mean) * jax.lax.rsqrt(var + 1e-5)
        ze = ze * gng_ref[g][None, :] + gnb_ref[g][None, :]

        dot = jnp.dot(ze, codexT_ref[g],
                      preferred_element_type=jnp.float32)
        zsq = jnp.sum(ze * ze, axis=1, keepdims=True)
        csq = jnp.sum(codexT_ref[g] ** 2, axis=0,
                      keepdims=True)
        d2 = zsq + csq - 2.0 * dot

        m = jnp.min(d2, axis=1, keepdims=True)
        kiota = jax.lax.broadcasted_iota(jnp.int32, (L, K), 1)
        idx = jnp.min(jnp.where(d2 == m, kiota, K), axis=1,
                      keepdims=True)
        probs = (kiota == idx).astype(jnp.float32)
        probs_ref[0, :, g * K:(g + 1) * K] = probs

        zq = jnp.dot(probs, codex_ref[g],
                     preferred_element_type=jnp.float32)
        diff = ze - zq
        loss_part = loss_part + jnp.sum(diff * diff)
        zq_parts.append(zq)

    @pl.when(b == 0)
    def _():
        loss_ref[...] = jnp.zeros((1, 1), jnp.float32)

    loss_ref[...] += jnp.reshape(loss_part, (1, 1)) * (
        (1.0 + BETA) / (B * L * D_CODEX))

    zq_full = jnp.concatenate(zq_parts, axis=1)
    mu = jnp.mean(zq_full, axis=1, keepdims=True)
    v = jnp.mean((zq_full - mu) ** 2, axis=1, keepdims=True)
    y = (zq_full - mu) * jax.lax.rsqrt(v + 1e-5)
    y = y * lng_ref[0][None, :] + lnb_ref[0][None, :]
    Zq_ref[0] = (jnp.dot(y, WpostT_ref[...],
                         preferred_element_type=jnp.float32)
                 + bpost_ref[0][None, :])


def kernel(Z, W_pre, gn_gamma, gn_beta, codex, ln_gamma, ln_beta,
           W_post, b_post):
    WpreT = W_pre.transpose(0, 2, 1)
    codexT = codex.transpose(0, 2, 1)
    WpostT = W_post.T
    gng = gn_gamma.reshape(G, DG)
    gnb = gn_beta.reshape(G, DG)
    lng = ln_gamma.reshape(1, D_CODEX)
    lnb = ln_beta.reshape(1, D_CODEX)
    bp = b_post.reshape(1, D_MODEL)

    full = lambda shape: pl.BlockSpec(shape, lambda b: (0,) * len(shape))
    zq_out, loss_out, probs_out = pl.pallas_call(
        _body,
        grid=(B,),
        in_specs=[
            pl.BlockSpec((1, L, D_MODEL), lambda b: (b, 0, 0)),
            full((G, DI, DG)),
            full((G, DG)),
            full((G, DG)),
            full((G, K, DG)),
            full((G, DG, K)),
            full((1, D_CODEX)),
            full((1, D_CODEX)),
            full((D_CODEX, D_MODEL)),
            full((1, D_MODEL)),
        ],
        out_specs=[
            pl.BlockSpec((1, L, D_MODEL), lambda b: (b, 0, 0)),
            pl.BlockSpec((1, 1), lambda b: (0, 0)),
            pl.BlockSpec((1, L, G * K), lambda b: (b, 0, 0)),
        ],
        out_shape=[
            jax.ShapeDtypeStruct((B, L, D_MODEL), jnp.float32),
            jax.ShapeDtypeStruct((1, 1), jnp.float32),
            jax.ShapeDtypeStruct((B, L, G * K), jnp.float32),
        ],
    )(Z, WpreT, gng, gnb, codex, codexT, lng, lnb, WpostT, bp)

    return (zq_out, loss_out[0, 0],
            probs_out.reshape(B, L, G, K))
